# Initial kernel scaffold; baseline (speedup 1.0000x reference)
import jax
import jax.numpy as jnp
from jax.experimental import pallas as pl


def kernel(x, edge_index, batch, W, b):
    raise NotImplementedError("write your pallas kernel here")


# speedup vs baseline: 34.5296x; 34.5296x over previous
import jax
import jax.numpy as jnp
from jax import lax
from jax.experimental import pallas as pl
from jax.experimental.pallas import tpu as pltpu
from jax.experimental.pallas import tpu_sc as plsc

N = 10000
NP = 10016
E = 320000
C = 16
G = 128
NCORES = 2
NSUB = 16
NTILES = NCORES * NSUB
CHUNK = 128
NCH = 79
EPT = NCH * CHUNK
EP = NTILES * EPT
RPT = NP // NSUB

_MESH = plsc.VectorSubcoreMesh(
    core_axis_name="c", subcore_axis_name="s",
    num_cores=NCORES, num_subcores=NSUB)


def _make_sc_body(with_gather):
    def body(*refs):
        if with_gather:
            (u_hbm, row_hbm, col_hbm, out_hbm,
             row_v, col_v, buf, zbuf, acc) = refs
        else:
            col_hbm, out_hbm, col_v, buf, zbuf, acc = refs
        cid = lax.axis_index("c")
        sid = lax.axis_index("s")
        gid = cid * NSUB + sid
        zeros16 = jnp.zeros((16,), jnp.float32)

        def zero_body(i, carry):
            zbuf[i, :] = zeros16
            return carry
        lax.fori_loop(0, RPT, zero_body, 0)
        pltpu.sync_copy(zbuf, acc.at[pl.ds(sid * RPT, RPT), :])
        if with_gather:
            pltpu.sync_copy(row_hbm.at[gid], row_v)
        else:
            ones16 = jnp.ones((16,), jnp.float32)

            def ones_body(i, carry):
                buf[i, :] = ones16
                return carry
            lax.fori_loop(0, CHUNK, ones_body, 0)
        pltpu.sync_copy(col_hbm.at[gid], col_v)
        plsc.subcore_barrier()

        def chunk_body(j, carry):
            if with_gather:
                pltpu.sync_copy(u_hbm.at[row_v.at[j]], buf)
            pltpu.sync_copy(buf, acc.at[col_v.at[j]], add=True)
            return carry
        lax.fori_loop(0, NCH, chunk_body, 0)
        plsc.subcore_barrier()
        pltpu.sync_copy(acc.at[pl.ds(sid * RPT, RPT), :],
                        out_hbm.at[cid, pl.ds(sid * RPT, RPT), :])
    return body


_round_kernel = pl.kernel(
    _make_sc_body(True),
    out_type=jax.ShapeDtypeStruct((NCORES, NP, C), jnp.float32),
    mesh=_MESH,
    compiler_params=pltpu.CompilerParams(use_tc_tiling_on_sc=False),
    scratch_types=[
        pltpu.VMEM((NCH, CHUNK), jnp.int32),
        pltpu.VMEM((NCH, CHUNK), jnp.int32),
        pltpu.VMEM((CHUNK, C), jnp.float32),
        pltpu.VMEM((RPT, C), jnp.float32),
        pltpu.VMEM_SHARED((NP, C), jnp.float32),
    ],
)

_deg_kernel = pl.kernel(
    _make_sc_body(False),
    out_type=jax.ShapeDtypeStruct((NCORES, NP, C), jnp.float32),
    mesh=_MESH,
    compiler_params=pltpu.CompilerParams(use_tc_tiling_on_sc=False),
    scratch_types=[
        pltpu.VMEM((NCH, CHUNK), jnp.int32),
        pltpu.VMEM((CHUNK, C), jnp.float32),
        pltpu.VMEM((RPT, C), jnp.float32),
        pltpu.VMEM_SHARED((NP, C), jnp.float32),
    ],
)


def _prep_body(x_ref, w_ref, degp_ref, u0_ref, dis_ref):
    deg = degp_ref[0, :, :1] + degp_ref[1, :, :1] + 1.0
    dis = lax.rsqrt(deg)
    y = jnp.dot(x_ref[...], w_ref[...], preferred_element_type=jnp.float32)
    discol = jnp.broadcast_to(dis, (NP, C))
    u0_ref[...] = discol * y
    dis_ref[...] = discol


def _prep(xp, W, degp):
    return pl.pallas_call(
        _prep_body,
        out_shape=(jax.ShapeDtypeStruct((NP, C), jnp.float32),
                   jax.ShapeDtypeStruct((NP, C), jnp.float32)),
    )(xp, W, degp)


def _mid_body(p_ref, u_ref, dis_ref, out_ref):
    d = dis_ref[...]
    out_ref[...] = d * d * (p_ref[0] + p_ref[1] + u_ref[...])


def _mid(p, u, dis):
    return pl.pallas_call(
        _mid_body,
        out_shape=jax.ShapeDtypeStruct((NP, C), jnp.float32),
    )(p, u, dis)


def _final_body(p_ref, u_ref, dis_ref, batch_ref, b_ref, out_ref):
    h2 = dis_ref[...] * (p_ref[0] + p_ref[1] + u_ref[...])
    gids = lax.broadcasted_iota(jnp.int32, (G, NP), 0)
    oh = (gids == batch_ref[...]).astype(jnp.float32)
    sums = jnp.dot(oh, h2, preferred_element_type=jnp.float32)
    cnt = jnp.sum(oh, axis=1, keepdims=True)
    mean = sums / jnp.maximum(cnt, 1.0) + b_ref[...] * jnp.minimum(cnt, 1.0)
    m = jnp.max(mean, axis=1, keepdims=True)
    lse = jnp.log(jnp.sum(jnp.exp(mean - m), axis=1, keepdims=True)) + m
    out_ref[...] = mean - lse


def _final(p, u, dis, batch2, b2):
    return pl.pallas_call(
        _final_body,
        out_shape=jax.ShapeDtypeStruct((G, C), jnp.float32),
    )(p, u, dis, batch2, b2)


def kernel(x, edge_index, batch, W, b):
    row = jnp.pad(edge_index[0], (0, EP - E),
                  constant_values=N).reshape(NTILES, NCH, CHUNK)
    col = jnp.pad(edge_index[1], (0, EP - E),
                  constant_values=N).reshape(NTILES, NCH, CHUNK)
    xp = jnp.pad(x, ((0, NP - N), (0, 0)))
    batch2 = jnp.pad(batch, (0, NP - N), constant_values=G).reshape(1, NP)
    b2 = b.reshape(1, C)

    degp = _deg_kernel(col)
    u0, dis = _prep(xp, W, degp)
    pA = _round_kernel(u0, row, col)
    u1 = _mid(pA, u0, dis)
    pB = _round_kernel(u1, row, col)
    return _final(pB, u1, dis, batch2, b2)


# speedup vs baseline: 39.8947x; 1.1554x over previous
import jax
import jax.numpy as jnp
from jax import lax
from jax.experimental import pallas as pl
from jax.experimental.pallas import tpu as pltpu
from jax.experimental.pallas import tpu_sc as plsc

N = 10000
NP = 10016
E = 320000
C = 16
G = 128
NCORES = 2
NSUB = 16
NTILES = NCORES * NSUB
CHUNK = 128
GSZ = 8
NG = 10
NCH = NG * GSZ
EPT = NCH * CHUNK
EP = NTILES * EPT
RPT = NP // NSUB

_MESH = plsc.VectorSubcoreMesh(
    core_axis_name="c", subcore_axis_name="s",
    num_cores=NCORES, num_subcores=NSUB)


def _make_sc_body(with_gather):
    def body(*refs):
        if with_gather:
            (u_hbm, row_hbm, col_hbm, out_hbm,
             row_v, col_v, buf, zbuf, acc, gsem, ssem) = refs
        else:
            col_hbm, out_hbm, col_v, buf, zbuf, acc, ssem = refs
        cid = lax.axis_index("c")
        sid = lax.axis_index("s")
        gid = cid * NSUB + sid
        zeros16 = jnp.zeros((16,), jnp.float32)

        def zero_body(i, carry):
            zbuf[i, :] = zeros16
            return carry
        lax.fori_loop(0, RPT, zero_body, 0)
        pltpu.sync_copy(zbuf, acc.at[pl.ds(sid * RPT, RPT), :])
        if with_gather:
            pltpu.sync_copy(row_hbm.at[gid], row_v)
        else:
            ones16 = jnp.ones((16,), jnp.float32)

            def ones_body(i, carry):
                buf[i, :] = ones16
                return carry
            lax.fori_loop(0, CHUNK, ones_body, 0)
        pltpu.sync_copy(col_hbm.at[gid], col_v)
        plsc.subcore_barrier()

        if with_gather:
            def issue_gathers(g, par):
                for b in range(GSZ):
                    pltpu.async_copy(u_hbm.at[row_v.at[g * GSZ + b]],
                                     buf.at[par, b], gsem)

            def drain(sem, par):
                for b in range(GSZ):
                    pltpu.make_async_copy(
                        u_hbm.at[pl.ds(0, CHUNK), :], buf.at[par, b],
                        sem).wait()

            issue_gathers(0, 0)

            def g_body(g, carry):
                par = lax.rem(g, 2)
                drain(gsem, par)

                @pl.when(g + 1 < NG)
                def _():
                    issue_gathers(g + 1, 1 - par)
                for b in range(GSZ):
                    pltpu.async_copy(buf.at[par, b],
                                     acc.at[col_v.at[g * GSZ + b]],
                                     ssem, add=True)
                drain(ssem, par)
                return carry
            lax.fori_loop(0, NG, g_body, 0)
        else:
            def chunk_body(j, carry):
                pltpu.async_copy(buf, acc.at[col_v.at[j]], ssem, add=True)
                return carry
            lax.fori_loop(0, NCH, chunk_body, 0)

            def drain_body(j, carry):
                pltpu.make_async_copy(
                    buf, acc.at[pl.ds(0, CHUNK), :], ssem).wait()
                return carry
            lax.fori_loop(0, NCH, drain_body, 0)
        plsc.subcore_barrier()
        pltpu.sync_copy(acc.at[pl.ds(sid * RPT, RPT), :],
                        out_hbm.at[cid, pl.ds(sid * RPT, RPT), :])
    return body


_round_kernel = pl.kernel(
    _make_sc_body(True),
    out_type=jax.ShapeDtypeStruct((NCORES, NP, C), jnp.float32),
    mesh=_MESH,
    compiler_params=pltpu.CompilerParams(use_tc_tiling_on_sc=False),
    scratch_types=[
        pltpu.VMEM((NCH, CHUNK), jnp.int32),
        pltpu.VMEM((NCH, CHUNK), jnp.int32),
        pltpu.VMEM((2, GSZ, CHUNK, C), jnp.float32),
        pltpu.VMEM((RPT, C), jnp.float32),
        pltpu.VMEM_SHARED((NP, C), jnp.float32),
        pltpu.SemaphoreType.DMA,
        pltpu.SemaphoreType.DMA,
    ],
)

_deg_kernel = pl.kernel(
    _make_sc_body(False),
    out_type=jax.ShapeDtypeStruct((NCORES, NP, C), jnp.float32),
    mesh=_MESH,
    compiler_params=pltpu.CompilerParams(use_tc_tiling_on_sc=False),
    scratch_types=[
        pltpu.VMEM((NCH, CHUNK), jnp.int32),
        pltpu.VMEM((CHUNK, C), jnp.float32),
        pltpu.VMEM((RPT, C), jnp.float32),
        pltpu.VMEM_SHARED((NP, C), jnp.float32),
        pltpu.SemaphoreType.DMA,
    ],
)


def _prep_body(x_ref, w_ref, degp_ref, u0_ref, dis_ref):
    deg = degp_ref[0, :, :1] + degp_ref[1, :, :1] + 1.0
    dis = lax.rsqrt(deg)
    y = jnp.dot(x_ref[...], w_ref[...], preferred_element_type=jnp.float32)
    discol = jnp.broadcast_to(dis, (NP, C))
    u0_ref[...] = discol * y
    dis_ref[...] = discol


def _prep(xp, W, degp):
    return pl.pallas_call(
        _prep_body,
        out_shape=(jax.ShapeDtypeStruct((NP, C), jnp.float32),
                   jax.ShapeDtypeStruct((NP, C), jnp.float32)),
    )(xp, W, degp)


def _mid_body(p_ref, u_ref, dis_ref, out_ref):
    d = dis_ref[...]
    out_ref[...] = d * d * (p_ref[0] + p_ref[1] + u_ref[...])


def _mid(p, u, dis):
    return pl.pallas_call(
        _mid_body,
        out_shape=jax.ShapeDtypeStruct((NP, C), jnp.float32),
    )(p, u, dis)


def _final_body(p_ref, u_ref, dis_ref, batch_ref, b_ref, out_ref):
    h2 = dis_ref[...] * (p_ref[0] + p_ref[1] + u_ref[...])
    gids = lax.broadcasted_iota(jnp.int32, (G, NP), 0)
    oh = (gids == batch_ref[...]).astype(jnp.float32)
    sums = jnp.dot(oh, h2, preferred_element_type=jnp.float32)
    cnt = jnp.sum(oh, axis=1, keepdims=True)
    mean = sums / jnp.maximum(cnt, 1.0) + b_ref[...] * jnp.minimum(cnt, 1.0)
    m = jnp.max(mean, axis=1, keepdims=True)
    lse = jnp.log(jnp.sum(jnp.exp(mean - m), axis=1, keepdims=True)) + m
    out_ref[...] = mean - lse


def _final(p, u, dis, batch2, b2):
    return pl.pallas_call(
        _final_body,
        out_shape=jax.ShapeDtypeStruct((G, C), jnp.float32),
    )(p, u, dis, batch2, b2)


def kernel(x, edge_index, batch, W, b):
    row = jnp.pad(edge_index[0], (0, EP - E),
                  constant_values=N).reshape(NTILES, NCH, CHUNK)
    col = jnp.pad(edge_index[1], (0, EP - E),
                  constant_values=N).reshape(NTILES, NCH, CHUNK)
    xp = jnp.pad(x, ((0, NP - N), (0, 0)))
    batch2 = jnp.pad(batch, (0, NP - N), constant_values=G).reshape(1, NP)
    b2 = b.reshape(1, C)

    degp = _deg_kernel(col)
    u0, dis = _prep(xp, W, degp)
    pA = _round_kernel(u0, row, col)
    u1 = _mid(pA, u0, dis)
    pB = _round_kernel(u1, row, col)
    return _final(pB, u1, dis, batch2, b2)


# speedup vs baseline: 52.1560x; 1.3073x over previous
import jax
import jax.numpy as jnp
from jax import lax
from jax.experimental import pallas as pl
from jax.experimental.pallas import tpu as pltpu
from jax.experimental.pallas import tpu_sc as plsc

N = 10000
NP = 10016
E = 320000
C = 16
G = 128
NCORES = 2
NSUB = 16
NTILES = NCORES * NSUB
CHUNK = 128
CH_TOT = E // CHUNK
NCH = 79
PADCH = NTILES * NCH - CH_TOT
BTILE = CH_TOT - 31 * NCH
GSZ = 8
NGF = 9
TAIL = NCH - NGF * GSZ
RPT = NP // NSUB

_MESH = plsc.VectorSubcoreMesh(
    core_axis_name="c", subcore_axis_name="s",
    num_cores=NCORES, num_subcores=NSUB)


def _stage_indices(ei3, pad3, which, dst, gid):
    @pl.when(gid < NTILES - 1)
    def _():
        pltpu.sync_copy(ei3.at[which, pl.ds(gid * NCH, NCH)], dst)

    @pl.when(gid == NTILES - 1)
    def _():
        pltpu.sync_copy(ei3.at[which, pl.ds(31 * NCH, BTILE)],
                        dst.at[pl.ds(0, BTILE)])
        pltpu.sync_copy(pad3.at[which], dst.at[pl.ds(BTILE, PADCH)])


def _make_sc_body(with_gather):
    def body(*refs):
        if with_gather:
            (u_hbm, ei3, pad3, out_hbm,
             row_v, col_v, buf, zbuf, acc, gsem, ssem) = refs
        else:
            ei3, pad3, out_hbm, col_v, buf, zbuf, acc, ssem = refs
        cid = lax.axis_index("c")
        sid = lax.axis_index("s")
        gid = cid * NSUB + sid
        zeros16 = jnp.zeros((16,), jnp.float32)

        def zero_body(i, carry):
            zbuf[i, :] = zeros16
            return carry
        lax.fori_loop(0, RPT, zero_body, 0)
        pltpu.sync_copy(zbuf, acc.at[pl.ds(sid * RPT, RPT), :])
        if with_gather:
            _stage_indices(ei3, pad3, 0, row_v, gid)
        else:
            ones16 = jnp.ones((16,), jnp.float32)

            def ones_body(i, carry):
                buf[i, :] = ones16
                return carry
            lax.fori_loop(0, CHUNK, ones_body, 0)
        _stage_indices(ei3, pad3, 1, col_v, gid)
        plsc.subcore_barrier()

        if with_gather:
            def issue_g(g, par, size):
                for b in range(size):
                    pltpu.async_copy(u_hbm.at[row_v.at[g * GSZ + b]],
                                     buf.at[par, b], gsem)

            def issue_s(g, par, size):
                for b in range(size):
                    pltpu.async_copy(buf.at[par, b],
                                     acc.at[col_v.at[g * GSZ + b]],
                                     ssem, add=True)

            def drain(sem, k):
                for _ in range(k):
                    pltpu.make_async_copy(u_hbm.at[pl.ds(0, CHUNK), :],
                                          buf.at[0, 0], sem).wait()

            issue_g(0, 0, GSZ)

            def g_body(g, carry):
                par = lax.rem(g, 2)
                drain(gsem, GSZ)
                issue_s(g, par, GSZ)

                @pl.when(g >= 1)
                def _():
                    drain(ssem, GSZ)

                @pl.when(g + 1 < NGF)
                def _():
                    issue_g(g + 1, 1 - par, GSZ)
                return carry
            lax.fori_loop(0, NGF, g_body, 0)
            issue_g(NGF, 1, TAIL)
            drain(ssem, GSZ)
            drain(gsem, TAIL)
            issue_s(NGF, 1, TAIL)
            drain(ssem, TAIL)
        else:
            def chunk_body(j, carry):
                pltpu.async_copy(buf, acc.at[col_v.at[j]], ssem, add=True)
                return carry
            lax.fori_loop(0, NCH, chunk_body, 0)

            def drain_body(j, carry):
                pltpu.make_async_copy(
                    buf, acc.at[pl.ds(0, CHUNK), :], ssem).wait()
                return carry
            lax.fori_loop(0, NCH, drain_body, 0)
        plsc.subcore_barrier()
        pltpu.sync_copy(acc.at[pl.ds(sid * RPT, RPT), :],
                        out_hbm.at[cid, pl.ds(sid * RPT, RPT), :])
    return body


_round_kernel = pl.kernel(
    _make_sc_body(True),
    out_type=jax.ShapeDtypeStruct((NCORES, NP, C), jnp.float32),
    mesh=_MESH,
    compiler_params=pltpu.CompilerParams(use_tc_tiling_on_sc=False),
    scratch_types=[
        pltpu.VMEM((NCH, CHUNK), jnp.int32),
        pltpu.VMEM((NCH, CHUNK), jnp.int32),
        pltpu.VMEM((2, GSZ, CHUNK, C), jnp.float32),
        pltpu.VMEM((RPT, C), jnp.float32),
        pltpu.VMEM_SHARED((NP, C), jnp.float32),
        pltpu.SemaphoreType.DMA,
        pltpu.SemaphoreType.DMA,
    ],
)

_deg_kernel = pl.kernel(
    _make_sc_body(False),
    out_type=jax.ShapeDtypeStruct((NCORES, NP, C), jnp.float32),
    mesh=_MESH,
    compiler_params=pltpu.CompilerParams(use_tc_tiling_on_sc=False),
    scratch_types=[
        pltpu.VMEM((NCH, CHUNK), jnp.int32),
        pltpu.VMEM((CHUNK, C), jnp.float32),
        pltpu.VMEM((RPT, C), jnp.float32),
        pltpu.VMEM_SHARED((NP, C), jnp.float32),
        pltpu.SemaphoreType.DMA,
    ],
)


def _prep_body(x_ref, w_ref, degp_ref, u0_ref, dis_ref):
    deg = degp_ref[0, :, :1] + degp_ref[1, :, :1] + 1.0
    discol = jnp.broadcast_to(lax.rsqrt(deg), (NP, C))
    y = jnp.dot(x_ref[...], w_ref[...], preferred_element_type=jnp.float32)
    u0_ref[pl.ds(0, N), :] = discol[:N, :] * y
    u0_ref[pl.ds(N, NP - N), :] = jnp.zeros((NP - N, C), jnp.float32)
    dis_ref[...] = discol


def _prep(x, W, degp):
    return pl.pallas_call(
        _prep_body,
        out_shape=(jax.ShapeDtypeStruct((NP, C), jnp.float32),
                   jax.ShapeDtypeStruct((NP, C), jnp.float32)),
    )(x, W, degp)


def _mid_body(p_ref, u_ref, dis_ref, out_ref):
    d = dis_ref[...]
    out_ref[...] = d * d * (p_ref[0] + p_ref[1] + u_ref[...])


def _mid(p, u, dis):
    return pl.pallas_call(
        _mid_body,
        out_shape=jax.ShapeDtypeStruct((NP, C), jnp.float32),
    )(p, u, dis)


def _final_body(p_ref, u_ref, dis_ref, batch_ref, b_ref, out_ref):
    h2 = dis_ref[...] * (p_ref[0] + p_ref[1] + u_ref[...])
    gids = lax.broadcasted_iota(jnp.int32, (G, N), 0)
    oh = (gids == batch_ref[...]).astype(jnp.float32)
    sums = jnp.dot(oh, h2[:N, :], preferred_element_type=jnp.float32)
    cnt = jnp.sum(oh, axis=1, keepdims=True)
    mean = sums / jnp.maximum(cnt, 1.0) + b_ref[...] * jnp.minimum(cnt, 1.0)
    m = jnp.max(mean, axis=1, keepdims=True)
    lse = jnp.log(jnp.sum(jnp.exp(mean - m), axis=1, keepdims=True)) + m
    out_ref[...] = mean - lse


def _final(p, u, dis, batch2, b2):
    return pl.pallas_call(
        _final_body,
        out_shape=jax.ShapeDtypeStruct((G, C), jnp.float32),
    )(p, u, dis, batch2, b2)


def kernel(x, edge_index, batch, W, b):
    ei3 = edge_index.reshape(2, CH_TOT, CHUNK)
    pad3 = jnp.full((2, PADCH, CHUNK), N, jnp.int32)
    batch2 = batch.reshape(1, N)
    b2 = b.reshape(1, C)

    degp = _deg_kernel(ei3, pad3)
    u0, dis = _prep(x, W, degp)
    pA = _round_kernel(u0, ei3, pad3)
    u1 = _mid(pA, u0, dis)
    pB = _round_kernel(u1, ei3, pad3)
    return _final(pB, u1, dis, batch2, b2)


# speedup vs baseline: 64.0581x; 1.2282x over previous
import jax
import jax.numpy as jnp
from jax import lax
from jax.experimental import pallas as pl
from jax.experimental.pallas import tpu as pltpu
from jax.experimental.pallas import tpu_sc as plsc

N = 10000
NP = 10016
E = 320000
C = 16
G = 128
NCORES = 2
NSUB = 16
NTILES = NCORES * NSUB
CHUNK = 128
CH_TOT = E // CHUNK
NCH = 79
PADCH = NTILES * NCH - CH_TOT
BTILE = CH_TOT - 31 * NCH
GSZ = 8
NGF = 9
TAIL = NCH - NGF * GSZ
RPT = NP // NSUB

_MESH = plsc.VectorSubcoreMesh(
    core_axis_name="c", subcore_axis_name="s",
    num_cores=NCORES, num_subcores=NSUB)


def _stage_indices(ei3, pad3, which, dst, gid):
    @pl.when(gid < NTILES - 1)
    def _():
        pltpu.sync_copy(ei3.at[which, pl.ds(gid * NCH, NCH)], dst)

    @pl.when(gid == NTILES - 1)
    def _():
        pltpu.sync_copy(ei3.at[which, pl.ds(31 * NCH, BTILE)],
                        dst.at[pl.ds(0, BTILE)])
        pltpu.sync_copy(pad3.at[which], dst.at[pl.ds(BTILE, PADCH)])


def _make_sc_body(with_gather):
    def body(*refs):
        if with_gather:
            (u_hbm, ei3, pad3, out_hbm,
             row_v, col_v, buf, zbuf, acc, gsem, ssem) = refs
        else:
            ei3, pad3, out_hbm, col_v, buf, zbuf, acc, ssem = refs
        cid = lax.axis_index("c")
        sid = lax.axis_index("s")
        gid = cid * NSUB + sid
        zeros16 = jnp.zeros((16,), jnp.float32)

        def zero_body(i, carry):
            zbuf[i, :] = zeros16
            return carry
        lax.fori_loop(0, RPT, zero_body, 0)
        pltpu.sync_copy(zbuf, acc.at[pl.ds(sid * RPT, RPT), :])
        if with_gather:
            _stage_indices(ei3, pad3, 0, row_v, gid)
        else:
            ones16 = jnp.ones((16,), jnp.float32)

            def ones_body(i, carry):
                buf[i, :] = ones16
                return carry
            lax.fori_loop(0, CHUNK, ones_body, 0)
        _stage_indices(ei3, pad3, 1, col_v, gid)
        plsc.subcore_barrier()

        if with_gather:
            def issue_g(g, par, size):
                for b in range(size):
                    pltpu.async_copy(u_hbm.at[row_v.at[g * GSZ + b]],
                                     buf.at[par, b], gsem)

            def issue_s(g, par, size):
                for b in range(size):
                    pltpu.async_copy(buf.at[par, b],
                                     acc.at[col_v.at[g * GSZ + b]],
                                     ssem, add=True)

            def drain(sem, k):
                for _ in range(k):
                    pltpu.make_async_copy(u_hbm.at[pl.ds(0, CHUNK), :],
                                          buf.at[0, 0], sem).wait()

            issue_g(0, 0, GSZ)

            def g_body(g, carry):
                par = lax.rem(g, 2)
                drain(gsem, GSZ)
                issue_s(g, par, GSZ)

                @pl.when(g >= 1)
                def _():
                    drain(ssem, GSZ)

                @pl.when(g + 1 < NGF)
                def _():
                    issue_g(g + 1, 1 - par, GSZ)
                return carry
            lax.fori_loop(0, NGF, g_body, 0)
            issue_g(NGF, 1, TAIL)
            drain(ssem, GSZ)
            drain(gsem, TAIL)
            issue_s(NGF, 1, TAIL)
            drain(ssem, TAIL)
        else:
            def chunk_body(j, carry):
                pltpu.async_copy(buf, acc.at[col_v.at[j]], ssem, add=True)
                return carry
            lax.fori_loop(0, NCH, chunk_body, 0)

            def drain_body(j, carry):
                pltpu.make_async_copy(
                    buf, acc.at[pl.ds(0, CHUNK), :], ssem).wait()
                return carry
            lax.fori_loop(0, NCH, drain_body, 0)
        plsc.subcore_barrier()
        pltpu.sync_copy(acc.at[pl.ds(sid * RPT, RPT), :],
                        out_hbm.at[cid, pl.ds(sid * RPT, RPT), :])
    return body


_round_kernel = pl.kernel(
    _make_sc_body(True),
    out_type=jax.ShapeDtypeStruct((NCORES, NP, C), jnp.float32),
    mesh=_MESH,
    compiler_params=pltpu.CompilerParams(use_tc_tiling_on_sc=False),
    scratch_types=[
        pltpu.VMEM((NCH, CHUNK), jnp.int32),
        pltpu.VMEM((NCH, CHUNK), jnp.int32),
        pltpu.VMEM((2, GSZ, CHUNK, C), jnp.float32),
        pltpu.VMEM((RPT, C), jnp.float32),
        pltpu.VMEM_SHARED((NP, C), jnp.float32),
        pltpu.SemaphoreType.DMA,
        pltpu.SemaphoreType.DMA,
    ],
)

_deg_kernel = pl.kernel(
    _make_sc_body(False),
    out_type=jax.ShapeDtypeStruct((NCORES, NP, C), jnp.float32),
    mesh=_MESH,
    compiler_params=pltpu.CompilerParams(use_tc_tiling_on_sc=False),
    scratch_types=[
        pltpu.VMEM((NCH, CHUNK), jnp.int32),
        pltpu.VMEM((CHUNK, C), jnp.float32),
        pltpu.VMEM((RPT, C), jnp.float32),
        pltpu.VMEM_SHARED((NP, C), jnp.float32),
        pltpu.SemaphoreType.DMA,
    ],
)


PK = NP * C // 128
PKN = N * C // 128


def _prep_body(x8_ref, w_ref, degp_ref, u0_ref, dis_ref):
    dis = lax.rsqrt(degp_ref[0] + degp_ref[1] + 1.0)
    w = w_ref[...]
    blocks = []
    for j in range(8):
        parts = []
        if j:
            parts.append(jnp.zeros((128, C * j), jnp.float32))
        parts.append(w)
        if j < 7:
            parts.append(jnp.zeros((128, C * (7 - j)), jnp.float32))
        blocks.append(jnp.concatenate(parts, axis=1) if len(parts) > 1
                      else parts[0])
    wblk = jnp.concatenate(blocks, axis=0)
    ypk = jnp.dot(x8_ref[...], wblk, preferred_element_type=jnp.float32)
    u0_ref[pl.ds(0, PKN), :] = dis[:PKN, :] * ypk
    u0_ref[pl.ds(PKN, PK - PKN), :] = jnp.zeros((PK - PKN, 128), jnp.float32)
    dis_ref[...] = dis


def _prep(x8, W, degp_pk):
    return pl.pallas_call(
        _prep_body,
        out_shape=(jax.ShapeDtypeStruct((PK, 128), jnp.float32),
                   jax.ShapeDtypeStruct((PK, 128), jnp.float32)),
    )(x8, W, degp_pk)


def _mid_body(p_ref, u_ref, dis_ref, out_ref):
    d = dis_ref[...]
    out_ref[...] = d * d * (p_ref[0] + p_ref[1] + u_ref[...])


def _mid(p_pk, u_pk, dis_pk):
    return pl.pallas_call(
        _mid_body,
        out_shape=jax.ShapeDtypeStruct((PK, 128), jnp.float32),
    )(p_pk, u_pk, dis_pk)


def _final_body(p_ref, u_ref, dis_ref, batchj_ref, b_ref, out_ref):
    d = dis_ref[...]
    h2 = d * (p_ref[0] + p_ref[1] + u_ref[...])
    h2n = h2[:PKN, :]
    gids = lax.broadcasted_iota(jnp.int32, (G, PKN), 0)
    sums = jnp.zeros((G, C), jnp.float32)
    cnt = jnp.zeros((G, 1), jnp.float32)
    for j in range(8):
        oh = (gids == batchj_ref[j:j + 1, :]).astype(jnp.float32)
        sj = jnp.dot(oh, h2n, preferred_element_type=jnp.float32)
        sums = sums + sj[:, C * j:C * (j + 1)]
        cnt = cnt + jnp.sum(oh, axis=1, keepdims=True)
    mean = sums / jnp.maximum(cnt, 1.0) + b_ref[...] * jnp.minimum(cnt, 1.0)
    m = jnp.max(mean, axis=1, keepdims=True)
    lse = jnp.log(jnp.sum(jnp.exp(mean - m), axis=1, keepdims=True)) + m
    out_ref[...] = mean - lse


def _final(p_pk, u_pk, dis_pk, batchj, b2):
    return pl.pallas_call(
        _final_body,
        out_shape=jax.ShapeDtypeStruct((G, C), jnp.float32),
    )(p_pk, u_pk, dis_pk, batchj, b2)


def kernel(x, edge_index, batch, W, b):
    ei3 = edge_index.reshape(2, CH_TOT, CHUNK)
    pad3 = jnp.full((2, PADCH, CHUNK), N, jnp.int32)
    x8 = x.reshape(PKN, 1024)
    batchj = batch.reshape(PKN, 8).T
    b2 = b.reshape(1, C)

    degp = _deg_kernel(ei3, pad3)
    u0_pk, dis_pk = _prep(x8, W, degp.reshape(2, PK, 128))
    pA = _round_kernel(u0_pk.reshape(NP, C), ei3, pad3)
    u1_pk = _mid(pA.reshape(2, PK, 128), u0_pk, dis_pk)
    pB = _round_kernel(u1_pk.reshape(NP, C), ei3, pad3)
    return _final(pB.reshape(2, PK, 128), u1_pk, dis_pk, batchj, b2)


# speedup vs baseline: 64.6250x; 1.0088x over previous
import jax
import jax.numpy as jnp
from jax import lax
from jax.experimental import pallas as pl
from jax.experimental.pallas import tpu as pltpu
from jax.experimental.pallas import tpu_sc as plsc

N = 10000
NP = 10016
E = 320000
C = 16
G = 128
NCORES = 2
NSUB = 16
NTILES = NCORES * NSUB
CHUNK = 128
CH_TOT = E // CHUNK
NCH = 79
PADCH = NTILES * NCH - CH_TOT
BTILE = CH_TOT - 31 * NCH
GSZ = 16
NGF = NCH // GSZ
TAIL = NCH - NGF * GSZ
TAIL_PAR = NGF % 2
RPT = NP // NSUB

_MESH = plsc.VectorSubcoreMesh(
    core_axis_name="c", subcore_axis_name="s",
    num_cores=NCORES, num_subcores=NSUB)


def _stage_indices(ei3, pad3, which, dst, gid):
    @pl.when(gid < NTILES - 1)
    def _():
        pltpu.sync_copy(ei3.at[which, pl.ds(gid * NCH, NCH)], dst)

    @pl.when(gid == NTILES - 1)
    def _():
        pltpu.sync_copy(ei3.at[which, pl.ds(31 * NCH, BTILE)],
                        dst.at[pl.ds(0, BTILE)])
        pltpu.sync_copy(pad3.at[which], dst.at[pl.ds(BTILE, PADCH)])


def _make_sc_body(with_gather):
    def body(*refs):
        if with_gather:
            (u_hbm, ei3, pad3, out_hbm,
             row_v, col_v, buf, zbuf, acc, gsem, ssem) = refs
        else:
            ei3, pad3, out_hbm, col_v, buf, zbuf, acc, ssem = refs
        cid = lax.axis_index("c")
        sid = lax.axis_index("s")
        gid = cid * NSUB + sid
        zeros16 = jnp.zeros((16,), jnp.float32)

        def zero_body(i, carry):
            zbuf[i, :] = zeros16
            return carry
        lax.fori_loop(0, RPT, zero_body, 0)
        pltpu.sync_copy(zbuf, acc.at[pl.ds(sid * RPT, RPT), :])
        if with_gather:
            _stage_indices(ei3, pad3, 0, row_v, gid)
        else:
            ones16 = jnp.ones((16,), jnp.float32)

            def ones_body(i, carry):
                buf[i, :] = ones16
                return carry
            lax.fori_loop(0, CHUNK, ones_body, 0)
        _stage_indices(ei3, pad3, 1, col_v, gid)
        plsc.subcore_barrier()

        if with_gather:
            def issue_g(g, par, size):
                for b in range(size):
                    pltpu.async_copy(u_hbm.at[row_v.at[g * GSZ + b]],
                                     buf.at[par, b], gsem)

            def issue_s(g, par, size):
                for b in range(size):
                    pltpu.async_copy(buf.at[par, b],
                                     acc.at[col_v.at[g * GSZ + b]],
                                     ssem, add=True)

            def drain(sem, k):
                for _ in range(k):
                    pltpu.make_async_copy(u_hbm.at[pl.ds(0, CHUNK), :],
                                          buf.at[0, 0], sem).wait()

            issue_g(0, 0, GSZ)

            def g_body(g, carry):
                par = lax.rem(g, 2)
                drain(gsem, GSZ)
                issue_s(g, par, GSZ)

                @pl.when(g >= 1)
                def _():
                    drain(ssem, GSZ)

                @pl.when(g + 1 < NGF)
                def _():
                    issue_g(g + 1, 1 - par, GSZ)
                return carry
            lax.fori_loop(0, NGF, g_body, 0)
            issue_g(NGF, TAIL_PAR, TAIL)
            drain(ssem, GSZ)
            drain(gsem, TAIL)
            issue_s(NGF, TAIL_PAR, TAIL)
            drain(ssem, TAIL)
        else:
            def chunk_body(j, carry):
                pltpu.async_copy(buf, acc.at[col_v.at[j]], ssem, add=True)
                return carry
            lax.fori_loop(0, NCH, chunk_body, 0)

            def drain_body(j, carry):
                pltpu.make_async_copy(
                    buf, acc.at[pl.ds(0, CHUNK), :], ssem).wait()
                return carry
            lax.fori_loop(0, NCH, drain_body, 0)
        plsc.subcore_barrier()
        pltpu.sync_copy(acc.at[pl.ds(sid * RPT, RPT), :],
                        out_hbm.at[cid, pl.ds(sid * RPT, RPT), :])
    return body


_round_kernel = pl.kernel(
    _make_sc_body(True),
    out_type=jax.ShapeDtypeStruct((NCORES, NP, C), jnp.float32),
    mesh=_MESH,
    compiler_params=pltpu.CompilerParams(use_tc_tiling_on_sc=False),
    scratch_types=[
        pltpu.VMEM((NCH, CHUNK), jnp.int32),
        pltpu.VMEM((NCH, CHUNK), jnp.int32),
        pltpu.VMEM((2, GSZ, CHUNK, C), jnp.float32),
        pltpu.VMEM((RPT, C), jnp.float32),
        pltpu.VMEM_SHARED((NP, C), jnp.float32),
        pltpu.SemaphoreType.DMA,
        pltpu.SemaphoreType.DMA,
    ],
)

_deg_kernel = pl.kernel(
    _make_sc_body(False),
    out_type=jax.ShapeDtypeStruct((NCORES, NP, C), jnp.float32),
    mesh=_MESH,
    compiler_params=pltpu.CompilerParams(use_tc_tiling_on_sc=False),
    scratch_types=[
        pltpu.VMEM((NCH, CHUNK), jnp.int32),
        pltpu.VMEM((CHUNK, C), jnp.float32),
        pltpu.VMEM((RPT, C), jnp.float32),
        pltpu.VMEM_SHARED((NP, C), jnp.float32),
        pltpu.SemaphoreType.DMA,
    ],
)


PK = NP * C // 128
PKN = N * C // 128


def _prep_body(x8_ref, w_ref, degp_ref, u0_ref, dis_ref):
    dis = lax.rsqrt(degp_ref[0] + degp_ref[1] + 1.0)
    w = w_ref[...]
    blocks = []
    for j in range(8):
        parts = []
        if j:
            parts.append(jnp.zeros((128, C * j), jnp.float32))
        parts.append(w)
        if j < 7:
            parts.append(jnp.zeros((128, C * (7 - j)), jnp.float32))
        blocks.append(jnp.concatenate(parts, axis=1) if len(parts) > 1
                      else parts[0])
    wblk = jnp.concatenate(blocks, axis=0)
    ypk = jnp.dot(x8_ref[...], wblk, preferred_element_type=jnp.float32)
    u0_ref[pl.ds(0, PKN), :] = dis[:PKN, :] * ypk
    u0_ref[pl.ds(PKN, PK - PKN), :] = jnp.zeros((PK - PKN, 128), jnp.float32)
    dis_ref[...] = dis


def _prep(x8, W, degp_pk):
    return pl.pallas_call(
        _prep_body,
        out_shape=(jax.ShapeDtypeStruct((PK, 128), jnp.float32),
                   jax.ShapeDtypeStruct((PK, 128), jnp.float32)),
    )(x8, W, degp_pk)


def _mid_body(p_ref, u_ref, dis_ref, out_ref):
    d = dis_ref[...]
    out_ref[...] = d * d * (p_ref[0] + p_ref[1] + u_ref[...])


def _mid(p_pk, u_pk, dis_pk):
    return pl.pallas_call(
        _mid_body,
        out_shape=jax.ShapeDtypeStruct((PK, 128), jnp.float32),
    )(p_pk, u_pk, dis_pk)


def _final_body(p_ref, u_ref, dis_ref, batchj_ref, b_ref, out_ref):
    d = dis_ref[...]
    h2 = d * (p_ref[0] + p_ref[1] + u_ref[...])
    h2n = h2[:PKN, :]
    gids = lax.broadcasted_iota(jnp.int32, (G, PKN), 0)
    sums = jnp.zeros((G, C), jnp.float32)
    cnt = jnp.zeros((G, 1), jnp.float32)
    for j in range(8):
        oh = (gids == batchj_ref[j:j + 1, :]).astype(jnp.float32)
        sj = jnp.dot(oh, h2n, preferred_element_type=jnp.float32)
        sums = sums + sj[:, C * j:C * (j + 1)]
        cnt = cnt + jnp.sum(oh, axis=1, keepdims=True)
    mean = sums / jnp.maximum(cnt, 1.0) + b_ref[...] * jnp.minimum(cnt, 1.0)
    m = jnp.max(mean, axis=1, keepdims=True)
    lse = jnp.log(jnp.sum(jnp.exp(mean - m), axis=1, keepdims=True)) + m
    out_ref[...] = mean - lse


def _final(p_pk, u_pk, dis_pk, batchj, b2):
    return pl.pallas_call(
        _final_body,
        out_shape=jax.ShapeDtypeStruct((G, C), jnp.float32),
    )(p_pk, u_pk, dis_pk, batchj, b2)


def kernel(x, edge_index, batch, W, b):
    ei3 = edge_index.reshape(2, CH_TOT, CHUNK)
    pad3 = jnp.full((2, PADCH, CHUNK), N, jnp.int32)
    x8 = x.reshape(PKN, 1024)
    batchj = batch.reshape(PKN, 8).T
    b2 = b.reshape(1, C)

    degp = _deg_kernel(ei3, pad3)
    u0_pk, dis_pk = _prep(x8, W, degp.reshape(2, PK, 128))
    pA = _round_kernel(u0_pk.reshape(NP, C), ei3, pad3)
    u1_pk = _mid(pA.reshape(2, PK, 128), u0_pk, dis_pk)
    pB = _round_kernel(u1_pk.reshape(NP, C), ei3, pad3)
    return _final(pB.reshape(2, PK, 128), u1_pk, dis_pk, batchj, b2)


# speedup vs baseline: 67.6492x; 1.0468x over previous
import jax
import jax.numpy as jnp
from jax import lax
from jax.experimental import pallas as pl
from jax.experimental.pallas import tpu as pltpu
from jax.experimental.pallas import tpu_sc as plsc

N = 10000
NP = 10016
E = 320000
C = 16
G = 128
NCORES = 2
NSUB = 16
NTILES = NCORES * NSUB
CHUNK = 128
CH_TOT = E // CHUNK
NCH0 = 96
NCH1 = 62
NCHMAX = max(NCH0, NCH1)
C1BASE = NSUB * NCH0
T31BASE = C1BASE + 15 * NCH1
BT31 = CH_TOT - T31BASE
PADCH = T31BASE + NCH1 - CH_TOT
GSZ = 16
RPT = NP // NSUB

_MESH = plsc.VectorSubcoreMesh(
    core_axis_name="c", subcore_axis_name="s",
    num_cores=NCORES, num_subcores=NSUB)


def _stage_indices(ei3, pad3, which, dst, cid, sid):
    @pl.when(cid == 0)
    def _():
        pltpu.sync_copy(ei3.at[which, pl.ds(sid * NCH0, NCH0)],
                        dst.at[pl.ds(0, NCH0)])

    @pl.when((cid == 1) & (sid < NSUB - 1))
    def _():
        pltpu.sync_copy(ei3.at[which, pl.ds(C1BASE + sid * NCH1, NCH1)],
                        dst.at[pl.ds(0, NCH1)])

    @pl.when((cid == 1) & (sid == NSUB - 1))
    def _():
        pltpu.sync_copy(ei3.at[which, pl.ds(T31BASE, BT31)],
                        dst.at[pl.ds(0, BT31)])
        pltpu.sync_copy(pad3.at[which], dst.at[pl.ds(BT31, PADCH)])


def _make_sc_body(with_gather):
    def body(*refs):
        if with_gather:
            (u_hbm, ei3, pad3, out_hbm,
             row_v, col_v, buf, zbuf, acc, gsem, ssem) = refs
        else:
            ei3, pad3, out_hbm, col_v, buf, zbuf, acc, ssem = refs
        cid = lax.axis_index("c")
        sid = lax.axis_index("s")
        zeros16 = jnp.zeros((16,), jnp.float32)

        def zero_body(i, carry):
            zbuf[i, :] = zeros16
            return carry
        lax.fori_loop(0, RPT, zero_body, 0)
        pltpu.sync_copy(zbuf, acc.at[pl.ds(sid * RPT, RPT), :])
        if with_gather:
            _stage_indices(ei3, pad3, 0, row_v, cid, sid)
        else:
            ones16 = jnp.ones((16,), jnp.float32)

            def ones_body(i, carry):
                buf[i, :] = ones16
                return carry
            lax.fori_loop(0, CHUNK, ones_body, 0)
        _stage_indices(ei3, pad3, 1, col_v, cid, sid)
        plsc.subcore_barrier()

        if with_gather:
            def issue_g(g, par, size):
                for b in range(size):
                    pltpu.async_copy(u_hbm.at[row_v.at[g * GSZ + b]],
                                     buf.at[par, b], gsem)

            def issue_s(g, par, size):
                for b in range(size):
                    pltpu.async_copy(buf.at[par, b],
                                     acc.at[col_v.at[g * GSZ + b]],
                                     ssem, add=True)

            def drain(sem, k):
                for _ in range(k):
                    pltpu.make_async_copy(u_hbm.at[pl.ds(0, CHUNK), :],
                                          buf.at[0, 0], sem).wait()

            def pipeline(nch):
                ngf, tail = nch // GSZ, nch % GSZ
                tail_par = ngf % 2
                issue_g(0, 0, GSZ)

                def g_body(g, carry):
                    par = lax.rem(g, 2)
                    drain(gsem, GSZ)
                    issue_s(g, par, GSZ)

                    @pl.when(g >= 1)
                    def _():
                        drain(ssem, GSZ)

                    @pl.when(g + 1 < ngf)
                    def _():
                        issue_g(g + 1, 1 - par, GSZ)
                    return carry
                lax.fori_loop(0, ngf, g_body, 0)
                if tail:
                    issue_g(ngf, tail_par, tail)
                    drain(ssem, GSZ)
                    drain(gsem, tail)
                    issue_s(ngf, tail_par, tail)
                    drain(ssem, tail)
                else:
                    drain(ssem, GSZ)

            @pl.when(cid == 0)
            def _():
                pipeline(NCH0)

            @pl.when(cid == 1)
            def _():
                pipeline(NCH1)
        else:
            nch_t = jnp.where(cid == 0, NCH0, NCH1)

            def chunk_body(j, carry):
                pltpu.async_copy(buf, acc.at[col_v.at[j]], ssem, add=True)
                return carry
            lax.fori_loop(0, nch_t, chunk_body, 0)

            def drain_body(j, carry):
                pltpu.make_async_copy(
                    buf, acc.at[pl.ds(0, CHUNK), :], ssem).wait()
                return carry
            lax.fori_loop(0, nch_t, drain_body, 0)
        plsc.subcore_barrier()
        pltpu.sync_copy(acc.at[pl.ds(sid * RPT, RPT), :],
                        out_hbm.at[cid, pl.ds(sid * RPT, RPT), :])
    return body


_round_kernel = pl.kernel(
    _make_sc_body(True),
    out_type=jax.ShapeDtypeStruct((NCORES, NP, C), jnp.float32),
    mesh=_MESH,
    compiler_params=pltpu.CompilerParams(use_tc_tiling_on_sc=False),
    scratch_types=[
        pltpu.VMEM((NCHMAX, CHUNK), jnp.int32),
        pltpu.VMEM((NCHMAX, CHUNK), jnp.int32),
        pltpu.VMEM((2, GSZ, CHUNK, C), jnp.float32),
        pltpu.VMEM((RPT, C), jnp.float32),
        pltpu.VMEM_SHARED((NP, C), jnp.float32),
        pltpu.SemaphoreType.DMA,
        pltpu.SemaphoreType.DMA,
    ],
)

_deg_kernel = pl.kernel(
    _make_sc_body(False),
    out_type=jax.ShapeDtypeStruct((NCORES, NP, C), jnp.float32),
    mesh=_MESH,
    compiler_params=pltpu.CompilerParams(use_tc_tiling_on_sc=False),
    scratch_types=[
        pltpu.VMEM((NCHMAX, CHUNK), jnp.int32),
        pltpu.VMEM((CHUNK, C), jnp.float32),
        pltpu.VMEM((RPT, C), jnp.float32),
        pltpu.VMEM_SHARED((NP, C), jnp.float32),
        pltpu.SemaphoreType.DMA,
    ],
)


PK = NP * C // 128
PKN = N * C // 128


def _prep_body(x8_ref, w_ref, degp_ref, u0_ref, dis_ref):
    dis = lax.rsqrt(degp_ref[0] + degp_ref[1] + 1.0)
    w = w_ref[...]
    blocks = []
    for j in range(8):
        parts = []
        if j:
            parts.append(jnp.zeros((128, C * j), jnp.float32))
        parts.append(w)
        if j < 7:
            parts.append(jnp.zeros((128, C * (7 - j)), jnp.float32))
        blocks.append(jnp.concatenate(parts, axis=1) if len(parts) > 1
                      else parts[0])
    wblk = jnp.concatenate(blocks, axis=0)
    ypk = jnp.dot(x8_ref[...], wblk, preferred_element_type=jnp.float32)
    u0_ref[pl.ds(0, PKN), :] = dis[:PKN, :] * ypk
    u0_ref[pl.ds(PKN, PK - PKN), :] = jnp.zeros((PK - PKN, 128), jnp.float32)
    dis_ref[...] = dis


def _prep(x8, W, degp_pk):
    return pl.pallas_call(
        _prep_body,
        out_shape=(jax.ShapeDtypeStruct((PK, 128), jnp.float32),
                   jax.ShapeDtypeStruct((PK, 128), jnp.float32)),
    )(x8, W, degp_pk)


def _mid_body(p_ref, u_ref, dis_ref, out_ref):
    d = dis_ref[...]
    out_ref[...] = d * d * (p_ref[0] + p_ref[1] + u_ref[...])


def _mid(p_pk, u_pk, dis_pk):
    return pl.pallas_call(
        _mid_body,
        out_shape=jax.ShapeDtypeStruct((PK, 128), jnp.float32),
    )(p_pk, u_pk, dis_pk)


def _final_body(p_ref, u_ref, dis_ref, batchj_ref, b_ref, out_ref):
    d = dis_ref[...]
    h2 = d * (p_ref[0] + p_ref[1] + u_ref[...])
    h2n = h2[:PKN, :]
    gids = lax.broadcasted_iota(jnp.int32, (G, PKN), 0)
    sums = jnp.zeros((G, C), jnp.float32)
    cnt = jnp.zeros((G, 1), jnp.float32)
    for j in range(8):
        oh = (gids == batchj_ref[j:j + 1, :]).astype(jnp.float32)
        sj = jnp.dot(oh, h2n, preferred_element_type=jnp.float32)
        sums = sums + sj[:, C * j:C * (j + 1)]
        cnt = cnt + jnp.sum(oh, axis=1, keepdims=True)
    mean = sums / jnp.maximum(cnt, 1.0) + b_ref[...] * jnp.minimum(cnt, 1.0)
    m = jnp.max(mean, axis=1, keepdims=True)
    lse = jnp.log(jnp.sum(jnp.exp(mean - m), axis=1, keepdims=True)) + m
    out_ref[...] = mean - lse


def _final(p_pk, u_pk, dis_pk, batchj, b2):
    return pl.pallas_call(
        _final_body,
        out_shape=jax.ShapeDtypeStruct((G, C), jnp.float32),
    )(p_pk, u_pk, dis_pk, batchj, b2)


def kernel(x, edge_index, batch, W, b):
    ei3 = edge_index.reshape(2, CH_TOT, CHUNK)
    pad3 = jnp.full((2, PADCH, CHUNK), N, jnp.int32)
    x8 = x.reshape(PKN, 1024)
    batchj = batch.reshape(PKN, 8).T
    b2 = b.reshape(1, C)

    degp = _deg_kernel(ei3, pad3)
    u0_pk, dis_pk = _prep(x8, W, degp.reshape(2, PK, 128))
    pA = _round_kernel(u0_pk.reshape(NP, C), ei3, pad3)
    u1_pk = _mid(pA.reshape(2, PK, 128), u0_pk, dis_pk)
    pB = _round_kernel(u1_pk.reshape(NP, C), ei3, pad3)
    return _final(pB.reshape(2, PK, 128), u1_pk, dis_pk, batchj, b2)
